# baseline (speedup 1.0000x reference)
import dataclasses
import functools

import jax
import jax.numpy as jnp
from jax import lax
from jax.experimental import pallas as pl
from jax.experimental.pallas import tpu as pltpu
from jax.experimental.pallas import tpu_sc as plsc

N = 10000
NP = 10240
F_IN = 128
H1, C1 = 8, 16
D1 = H1 * C1
H2, C2 = 1, 64
E_RAW = 320000
E_LOOP = E_RAW + N
K = 64
NWORK = 32
CHUNKS_PER_WORKER = -(-E_LOOP // (K * NWORK))
EP = CHUNKS_PER_WORKER * K * NWORK
ROWS_PER_SUB = NP // 16
DROWS_PER_SUB = ROWS_PER_SUB // 16
BLK = 1024

_GD = lax.GatherDimensionNumbers(
    offset_dims=(), collapsed_slice_dims=(0,), start_index_map=(0,))


def _lane_gather(v, idx):
  return lax.gather(v, idx.reshape(16, 1), _GD, (1,),
                    mode=lax.GatherScatterMode.PROMISE_IN_BOUNDS)


def _lane_bcast(v, hd):
  return _lane_gather(v, jnp.full((16,), hd, dtype=jnp.int32))


def _pre1_body(x_ref, wt_ref, ams_ref, amd_ref, h_ref, as_ref, ad_ref):
  h = jnp.dot(x_ref[...], wt_ref[...], preferred_element_type=jnp.float32)
  h_ref[...] = h
  as_ref[...] = jnp.dot(h, ams_ref[...], preferred_element_type=jnp.float32)
  ad_ref[...] = jnp.dot(h, amd_ref[...], preferred_element_type=jnp.float32)


def _mid_body(acc_ref, den_ref, b1_ref, wt_ref, ams_ref, amd_ref,
              h2_ref, as_ref, ad_ref):
  acc = acc_ref[0] + acc_ref[1]
  den = den_ref[0] + den_ref[1]
  h = acc / (den + 1e-16) + b1_ref[...]
  h = jnp.where(h > 0, h, 0.2 * (jnp.exp(h) - 1.0))
  h2 = jnp.dot(h, wt_ref[...], preferred_element_type=jnp.float32)
  h2_ref[:, :C2] = h2
  h2_ref[:, C2:] = jnp.zeros_like(h2)
  as_ref[...] = jnp.dot(h2, ams_ref[...], preferred_element_type=jnp.float32)
  ad_ref[...] = jnp.dot(h2, amd_ref[...], preferred_element_type=jnp.float32)


def _fin_body(acc_ref, den_ref, b2_ref, out_ref):
  acc = acc_ref[0] + acc_ref[1]
  den = den_ref[0] + den_ref[1]
  out_ref[...] = acc[:, :C2] / (den[:, :C2] + 1e-16) + b2_ref[...]


def _make_edge_pass(nheads):
  head_of = [min(j, nheads - 1) for j in range(8)]
  mesh = plsc.VectorSubcoreMesh(core_axis_name="c", subcore_axis_name="s")
  cp = pltpu.CompilerParams()
  if "needs_layout_passes" in pltpu.CompilerParams.__dataclass_fields__:
    cp = dataclasses.replace(cp, needs_layout_passes=False)

  @functools.partial(
      pl.kernel,
      out_type=(jax.ShapeDtypeStruct((2, NP, 128), jnp.float32),
                jax.ShapeDtypeStruct((2, NP, 128), jnp.float32)),
      mesh=mesh,
      compiler_params=cp,
      scratch_types=[
          pltpu.VMEM((K,), jnp.int32),
          pltpu.VMEM((K,), jnp.int32),
          pltpu.VMEM((K,), jnp.int32),
          pltpu.VMEM((K, 128), jnp.float32),
          pltpu.VMEM((K, 128), jnp.float32),
          pltpu.VMEM((K, 128), jnp.float32),
          pltpu.VMEM((K, 128), jnp.float32),
          pltpu.VMEM_SHARED((NP, 128), jnp.float32),
          pltpu.VMEM_SHARED((NP // 16, 128), jnp.float32),
          pltpu.SemaphoreType.DMA,
          pltpu.SemaphoreType.DMA,
          pltpu.SemaphoreType.DMA,
      ],
  )
  def edge_pass(h_hbm, as_hbm, ad_hbm, src_hbm, dst_hbm, acc_hbm, den_hbm,
                sidx, didx, didx16, as_b, ad_b, h_b, w_b, acc_sh, den_sh,
                sem0, sem1, sem2):
    cid = lax.axis_index("c")
    sid = lax.axis_index("s")
    wid = cid * 16 + sid
    lane = lax.iota(jnp.int32, 16)
    zero16 = jnp.zeros((16,), jnp.float32)

    @pl.loop(0, K)
    def _(i):
      @pl.loop(0, 128, step=16)
      def _(j):
        h_b[i, pl.ds(j, 16)] = zero16
        w_b[i, pl.ds(j, 16)] = zero16

    row0 = sid * ROWS_PER_SUB
    drow0 = sid * DROWS_PER_SUB

    @pl.loop(0, ROWS_PER_SUB, step=K)
    def _(r):
      pltpu.sync_copy(h_b, acc_sh.at[pl.ds(row0 + r, K)])

    pltpu.sync_copy(w_b.at[pl.ds(0, DROWS_PER_SUB)],
                    den_sh.at[pl.ds(drow0, DROWS_PER_SUB)])

    plsc.subcore_barrier()

    @pl.loop(0, CHUNKS_PER_WORKER)
    def _(g):
      base = (wid * CHUNKS_PER_WORKER + g) * K
      pltpu.sync_copy(src_hbm.at[pl.ds(base, K)], sidx)
      pltpu.sync_copy(dst_hbm.at[pl.ds(base, K)], didx)
      cp0 = pltpu.async_copy(as_hbm.at[sidx], as_b, sem0)
      cp1 = pltpu.async_copy(ad_hbm.at[didx], ad_b, sem1)
      cp2 = pltpu.async_copy(h_hbm.at[sidx], h_b, sem2)

      @pl.loop(0, K, step=16)
      def _(i):
        didx16[pl.ds(i, 16)] = lax.shift_right_logical(didx[pl.ds(i, 16)], 4)

      cp0.wait()
      cp1.wait()
      cp2.wait()

      @pl.loop(0, K)
      def _(e):
        dv = plsc.load_gather(didx, [jnp.full((16,), e, jnp.int32)])
        deq = lane == (dv & 15)
        for j in range(8):
          sl = pl.ds(j * 16, 16)
          al = as_b[e, sl] + ad_b[e, sl]
          al = jnp.where(al > 0, al, al * 0.2)
          wj = jnp.exp(al)
          h_b[e, sl] = h_b[e, sl] * wj
          if head_of[j] == j:
            w_b[e, sl] = jnp.where(deq, wj, 0.0)

      pltpu.sync_copy(h_b, acc_sh.at[didx], add=True)
      pltpu.sync_copy(w_b, den_sh.at[didx16], add=True)

    plsc.subcore_barrier()

    pltpu.sync_copy(acc_sh.at[pl.ds(row0, ROWS_PER_SUB)],
                    acc_hbm.at[cid].at[pl.ds(row0, ROWS_PER_SUB)])

    pltpu.sync_copy(den_sh.at[pl.ds(drow0, DROWS_PER_SUB)],
                    ad_b.at[pl.ds(0, DROWS_PER_SUB)])

    @pl.loop(0, ROWS_PER_SUB, step=K)
    def _(t):
      @pl.loop(0, K)
      def _(u):
        nl = t + u
        r = lax.shift_right_logical(nl, 4)
        m = jnp.full((16,), nl & 15, jnp.int32)
        for j in range(8):
          q = ad_b[r, pl.ds(head_of[j] * 16, 16)]
          h_b[u, pl.ds(j * 16, 16)] = _lane_gather(q, m)

      pltpu.sync_copy(h_b, den_hbm.at[cid].at[pl.ds(row0 + t, K)])

  return edge_pass


_edge_pass1 = _make_edge_pass(H1)
_edge_pass2 = _make_edge_pass(H2)


def _att_mat(att, D):
  nheads = att.shape[1]
  cdim = D // nheads
  row_head = jnp.arange(D) // cdim
  lane_head = jnp.minimum(jnp.arange(128) // 16, nheads - 1)
  mask = (row_head[:, None] == lane_head[None, :]).astype(jnp.float32)
  return mask * att.reshape(D, 1)


def kernel(x, edge_index, W1, att_src1, att_dst1, b1, W2, att_src2, att_dst2,
           b2):
  loop = jnp.arange(N, dtype=edge_index.dtype)
  src = jnp.concatenate([edge_index[0], loop]).astype(jnp.int32)
  dst = jnp.concatenate([edge_index[1], loop]).astype(jnp.int32)
  pad = jnp.full((EP - E_LOOP,), N, jnp.int32)
  src = jnp.concatenate([src, pad])
  dst = jnp.concatenate([dst, pad])

  x_p = jnp.pad(x, ((0, NP - N), (0, 0)))
  ams1 = _att_mat(att_src1, D1)
  amd1 = _att_mat(att_dst1, D1)
  ams2 = _att_mat(att_src2, C2)
  amd2 = _att_mat(att_dst2, C2)

  grid = NP // BLK
  row_spec = lambda w: pl.BlockSpec((BLK, w), lambda i: (i, 0))
  full_spec = lambda a, b: pl.BlockSpec((a, b), lambda i: (0, 0))
  pair_spec = pl.BlockSpec((2, BLK, 128), lambda i: (0, i, 0))

  h1, as1, ad1 = pl.pallas_call(
      _pre1_body,
      grid=(grid,),
      in_specs=[row_spec(F_IN), full_spec(F_IN, D1), full_spec(D1, 128),
                full_spec(D1, 128)],
      out_specs=[row_spec(D1), row_spec(128), row_spec(128)],
      out_shape=[jax.ShapeDtypeStruct((NP, D1), jnp.float32),
                 jax.ShapeDtypeStruct((NP, 128), jnp.float32),
                 jax.ShapeDtypeStruct((NP, 128), jnp.float32)],
  )(x_p, W1.T, ams1, amd1)

  acc1, den1 = _edge_pass1(h1, as1, ad1, src, dst)

  h2, as2, ad2 = pl.pallas_call(
      _mid_body,
      grid=(grid,),
      in_specs=[pair_spec, pair_spec, full_spec(1, D1), full_spec(D1, C2),
                full_spec(C2, 128), full_spec(C2, 128)],
      out_specs=[row_spec(128), row_spec(128), row_spec(128)],
      out_shape=[jax.ShapeDtypeStruct((NP, 128), jnp.float32),
                 jax.ShapeDtypeStruct((NP, 128), jnp.float32),
                 jax.ShapeDtypeStruct((NP, 128), jnp.float32)],
  )(acc1, den1, b1.reshape(1, D1), W2.T, ams2, amd2)

  acc2, den2 = _edge_pass2(h2, as2, ad2, src, dst)

  out = pl.pallas_call(
      _fin_body,
      grid=(grid,),
      in_specs=[pair_spec, pair_spec, full_spec(1, C2)],
      out_specs=pl.BlockSpec((BLK, C2), lambda i: (i, 0)),
      out_shape=jax.ShapeDtypeStruct((NP, C2), jnp.float32),
  )(acc2, den2, b2.reshape(1, C2))

  return out[:N]


# speedup vs baseline: 2.8208x; 2.8208x over previous
import dataclasses
import functools

import jax
import jax.numpy as jnp
from jax import lax
from jax.experimental import pallas as pl
from jax.experimental.pallas import tpu as pltpu
from jax.experimental.pallas import tpu_sc as plsc

N = 10000
NP = 10240
F_IN = 128
H1, C1 = 8, 16
D1 = H1 * C1
H2, C2 = 1, 64
E_RAW = 320000
E_LOOP = E_RAW + N
K = 40
NWORK = 32
CHUNKS = 259
EP = CHUNKS * K * NWORK
EP_ALLOC = EP + 2 * K
ROWS_PER_SUB = NP // 16
DROWS_PER_SUB = ROWS_PER_SUB // 16
BLK = 1024

_GD = lax.GatherDimensionNumbers(
    offset_dims=(), collapsed_slice_dims=(0,), start_index_map=(0,))


def _lane_gather(v, idx):
  return lax.gather(v, idx.reshape(16, 1), _GD, (1,),
                    mode=lax.GatherScatterMode.PROMISE_IN_BOUNDS)


def _lane_bcast(v, hd):
  return _lane_gather(v, jnp.full((16,), hd, dtype=jnp.int32))


def _pre1_body(x_ref, wt_ref, ams_ref, amd_ref, h_ref, as_ref, ad_ref):
  h = jnp.dot(x_ref[...], wt_ref[...], preferred_element_type=jnp.float32)
  h_ref[...] = h
  as_ref[...] = jnp.dot(h, ams_ref[...], preferred_element_type=jnp.float32)
  ad_ref[...] = jnp.dot(h, amd_ref[...], preferred_element_type=jnp.float32)


def _mid_body(acc_ref, den_ref, b1_ref, wt_ref, ams_ref, amd_ref,
              h2_ref, as_ref, ad_ref):
  acc = acc_ref[0] + acc_ref[1]
  den = den_ref[0] + den_ref[1]
  h = acc / (den + 1e-16) + b1_ref[...]
  h = jnp.where(h > 0, h, 0.2 * (jnp.exp(h) - 1.0))
  h2 = jnp.dot(h, wt_ref[...], preferred_element_type=jnp.float32)
  h2_ref[:, :C2] = h2
  h2_ref[:, C2:] = jnp.zeros_like(h2)
  as_ref[...] = jnp.dot(h2, ams_ref[...], preferred_element_type=jnp.float32)
  ad_ref[...] = jnp.dot(h2, amd_ref[...], preferred_element_type=jnp.float32)


def _fin_body(acc_ref, den_ref, b2_ref, out_ref):
  acc = acc_ref[0] + acc_ref[1]
  den = den_ref[0] + den_ref[1]
  out_ref[...] = acc[:, :C2] / (den[:, :C2] + 1e-16) + b2_ref[...]


def _make_edge_pass(nheads):
  head_of = [min(j, nheads - 1) for j in range(8)]
  mesh = plsc.VectorSubcoreMesh(core_axis_name="c", subcore_axis_name="s")
  cp = pltpu.CompilerParams()
  if "needs_layout_passes" in pltpu.CompilerParams.__dataclass_fields__:
    cp = dataclasses.replace(cp, needs_layout_passes=False)

  @functools.partial(
      pl.kernel,
      out_type=(jax.ShapeDtypeStruct((2, NP, 128), jnp.float32),
                jax.ShapeDtypeStruct((2, NP, 128), jnp.float32)),
      mesh=mesh,
      compiler_params=cp,
      scratch_types=(
          [pltpu.VMEM((K,), jnp.int32)] * 6 +
          [pltpu.VMEM((K,), jnp.int32)] * 2 +
          [pltpu.VMEM((K, 128), jnp.float32)] * 8 +
          [pltpu.VMEM_SHARED((NP, 128), jnp.float32),
           pltpu.VMEM_SHARED((NP // 16, 128), jnp.float32)] +
          [pltpu.SemaphoreType.DMA] * 5
      ),
  )
  def edge_pass(h_hbm, as_hbm, ad_hbm, src_hbm, dst_hbm, acc_hbm, den_hbm,
                si0, si1, si2, di0, di1, di2, d16a, d16b,
                as0, as1, ad0, ad1, hb0, hb1, wb0, wb1,
                acc_sh, den_sh, is0, is1, is2, gs0, gs1):
    SIDX, DIDX, D16 = [si0, si1, si2], [di0, di1, di2], [d16a, d16b]
    AS, AD, HB, WB = [as0, as1], [ad0, ad1], [hb0, hb1], [wb0, wb1]
    ISEM, GSEM = [is0, is1, is2], [gs0, gs1]

    cid = lax.axis_index("c")
    sid = lax.axis_index("s")
    wid = cid * 16 + sid
    lane = lax.iota(jnp.int32, 16)
    zero16 = jnp.zeros((16,), jnp.float32)

    def issue_idx(cidx, ib):
      base = (wid * CHUNKS + cidx) * K
      pltpu.async_copy(src_hbm.at[pl.ds(base, K)], SIDX[ib], ISEM[ib])
      pltpu.async_copy(dst_hbm.at[pl.ds(base, K)], DIDX[ib], ISEM[ib])

    def wait_idx(ib):
      pltpu.make_async_copy(src_hbm.at[pl.ds(0, K)], SIDX[ib], ISEM[ib]).wait()
      pltpu.make_async_copy(dst_hbm.at[pl.ds(0, K)], DIDX[ib], ISEM[ib]).wait()

    def issue_gathers(db, ib):
      pltpu.async_copy(as_hbm.at[SIDX[ib]], AS[db], GSEM[db])
      pltpu.async_copy(ad_hbm.at[DIDX[ib]], AD[db], GSEM[db])
      pltpu.async_copy(h_hbm.at[SIDX[ib]], HB[db], GSEM[db])

    def wait_gathers(db, ib):
      pltpu.make_async_copy(as_hbm.at[SIDX[ib]], AS[db], GSEM[db]).wait()
      pltpu.make_async_copy(ad_hbm.at[DIDX[ib]], AD[db], GSEM[db]).wait()
      pltpu.make_async_copy(h_hbm.at[SIDX[ib]], HB[db], GSEM[db]).wait()

    def sync_scatters(db, ib):
      @pl.loop(0, K, step=16)
      def _(i):
        D16[db][pl.ds(i, 16)] = lax.shift_right_logical(
            DIDX[ib][pl.ds(i, 16)], 4)
      pltpu.sync_copy(HB[db], acc_sh.at[DIDX[ib]], add=True)
      pltpu.sync_copy(WB[db], den_sh.at[D16[db]], add=True)

    def compute(db, ib):
      as_b, ad_b, h_b, w_b, didx = AS[db], AD[db], HB[db], WB[db], DIDX[ib]

      @pl.loop(0, K)
      def _(e):
        al = as_b[e, pl.ds(0, 16)] + ad_b[e, pl.ds(0, 16)]
        al = jnp.where(al > 0, al, al * 0.2)
        w = jnp.exp(al)
        dv = plsc.load_gather(didx, [jnp.full((16,), e, jnp.int32)])
        deq = lane == (dv & 15)
        for j in range(8):
          sl = pl.ds(j * 16, 16)
          wb = _lane_bcast(w, head_of[j])
          h_b[e, sl] = h_b[e, sl] * wb
          if head_of[j] == j:
            w_b[e, sl] = jnp.where(deq, wb, 0.0)

    @pl.loop(0, K)
    def _(i):
      @pl.loop(0, 128, step=16)
      def _(j):
        hb0[i, pl.ds(j, 16)] = zero16
        wb0[i, pl.ds(j, 16)] = zero16
        wb1[i, pl.ds(j, 16)] = zero16

    row0 = sid * ROWS_PER_SUB
    drow0 = sid * DROWS_PER_SUB

    @pl.loop(0, ROWS_PER_SUB, step=K)
    def _(r):
      pltpu.sync_copy(hb0, acc_sh.at[pl.ds(row0 + r, K)])

    pltpu.sync_copy(wb0.at[pl.ds(0, DROWS_PER_SUB)],
                    den_sh.at[pl.ds(drow0, DROWS_PER_SUB)])

    plsc.subcore_barrier()

    issue_idx(0, 0)
    issue_idx(1, 1)
    wait_idx(0)
    issue_gathers(0, 0)

    wait_gathers(0, 0)
    issue_idx(2, 2)
    wait_idx(1)
    issue_gathers(1, 1)
    compute(0, 0)
    sync_scatters(0, 0)

    @pl.loop(0, (CHUNKS - 1) // 6)
    def _(t):
      g0 = 1 + t * 6
      for u in range(6):
        g = g0 + u
        db, dp = (1 + u) & 1, u & 1
        ib_g, ib_n1, ib_n2 = (1 + u) % 3, (2 + u) % 3, u % 3
        wait_gathers(db, ib_g)
        issue_idx(g + 2, ib_n2)
        wait_idx(ib_n1)
        issue_gathers(dp, ib_n1)
        compute(db, ib_g)
        sync_scatters(db, ib_g)

    wait_gathers(1, 1)
    wait_idx(2)
    plsc.subcore_barrier()

    pltpu.sync_copy(acc_sh.at[pl.ds(row0, ROWS_PER_SUB)],
                    acc_hbm.at[cid].at[pl.ds(row0, ROWS_PER_SUB)])

    pltpu.sync_copy(den_sh.at[pl.ds(drow0, DROWS_PER_SUB)],
                    ad0.at[pl.ds(0, DROWS_PER_SUB)])

    @pl.loop(0, ROWS_PER_SUB, step=K)
    def _(r):
      @pl.loop(0, K)
      def _(u):
        nl = r + u
        rr = lax.shift_right_logical(nl, 4)
        m = jnp.full((16,), nl & 15, jnp.int32)
        for j in range(8):
          q = ad0[rr, pl.ds(head_of[j] * 16, 16)]
          hb0[u, pl.ds(j * 16, 16)] = _lane_gather(q, m)

      pltpu.sync_copy(hb0, den_hbm.at[cid].at[pl.ds(row0 + r, K)])

  return edge_pass


_edge_pass1 = _make_edge_pass(H1)
_edge_pass2 = _make_edge_pass(H2)


def _att_mat(att, D):
  nheads = att.shape[1]
  cdim = D // nheads
  cols = jnp.repeat(jnp.arange(nheads, dtype=jnp.int32), cdim)
  m = jnp.zeros((D, 128), jnp.float32)
  return m.at[jnp.arange(D), cols].set(att.reshape(D))


def kernel(x, edge_index, W1, att_src1, att_dst1, b1, W2, att_src2, att_dst2,
           b2):
  loop = jnp.arange(N, dtype=edge_index.dtype)
  src = jnp.concatenate([edge_index[0], loop]).astype(jnp.int32)
  dst = jnp.concatenate([edge_index[1], loop]).astype(jnp.int32)
  pad = jnp.full((EP_ALLOC - E_LOOP,), N, jnp.int32)
  src = jnp.concatenate([src, pad])
  dst = jnp.concatenate([dst, pad])

  x_p = jnp.pad(x, ((0, NP - N), (0, 0)))
  ams1 = _att_mat(att_src1, D1)
  amd1 = _att_mat(att_dst1, D1)
  ams2 = _att_mat(att_src2, C2)
  amd2 = _att_mat(att_dst2, C2)

  grid = NP // BLK
  row_spec = lambda w: pl.BlockSpec((BLK, w), lambda i: (i, 0))
  full_spec = lambda a, b: pl.BlockSpec((a, b), lambda i: (0, 0))
  pair_spec = pl.BlockSpec((2, BLK, 128), lambda i: (0, i, 0))

  h1, as1, ad1 = pl.pallas_call(
      _pre1_body,
      grid=(grid,),
      in_specs=[row_spec(F_IN), full_spec(F_IN, D1), full_spec(D1, 128),
                full_spec(D1, 128)],
      out_specs=[row_spec(D1), row_spec(128), row_spec(128)],
      out_shape=[jax.ShapeDtypeStruct((NP, D1), jnp.float32),
                 jax.ShapeDtypeStruct((NP, 128), jnp.float32),
                 jax.ShapeDtypeStruct((NP, 128), jnp.float32)],
  )(x_p, W1.T, ams1, amd1)

  acc1, den1 = _edge_pass1(h1, as1, ad1, src, dst)

  h2, as2, ad2 = pl.pallas_call(
      _mid_body,
      grid=(grid,),
      in_specs=[pair_spec, pair_spec, full_spec(1, D1), full_spec(D1, C2),
                full_spec(C2, 128), full_spec(C2, 128)],
      out_specs=[row_spec(128), row_spec(128), row_spec(128)],
      out_shape=[jax.ShapeDtypeStruct((NP, 128), jnp.float32),
                 jax.ShapeDtypeStruct((NP, 128), jnp.float32),
                 jax.ShapeDtypeStruct((NP, 128), jnp.float32)],
  )(acc1, den1, b1.reshape(1, D1), W2.T, ams2, amd2)

  acc2, den2 = _edge_pass2(h2, as2, ad2, src, dst)

  out = pl.pallas_call(
      _fin_body,
      grid=(grid,),
      in_specs=[pair_spec, pair_spec, full_spec(1, C2)],
      out_specs=pl.BlockSpec((BLK, C2), lambda i: (i, 0)),
      out_shape=jax.ShapeDtypeStruct((NP, C2), jnp.float32),
  )(acc2, den2, b2.reshape(1, C2))

  return out[:N]


# speedup vs baseline: 3.6639x; 1.2989x over previous
import dataclasses
import functools

import jax
import jax.numpy as jnp
from jax import lax
from jax.experimental import pallas as pl
from jax.experimental.pallas import tpu as pltpu
from jax.experimental.pallas import tpu_sc as plsc

N = 10000
NP = 10240
F_IN = 128
H1, C1 = 8, 16
D1 = H1 * C1
H2, C2 = 1, 64
E_RAW = 320000
E_LOOP = E_RAW + N
K = 40
NWORK = 32
CHUNKS = 259
EP = CHUNKS * K * NWORK
EP_ALLOC = EP + 2 * K
ROWS_PER_SUB = NP // 16
DROWS_PER_SUB = ROWS_PER_SUB // 16
NP_ACC = NP + NP // 16
BLK = 1024
_W16 = (0, 16, 24)

_GD = lax.GatherDimensionNumbers(
    offset_dims=(), collapsed_slice_dims=(0,), start_index_map=(0,))


def _lane_gather(v, idx):
  return lax.gather(v, idx.reshape(16, 1), _GD, (1,),
                    mode=lax.GatherScatterMode.PROMISE_IN_BOUNDS)


def _lane_bcast(v, hd):
  return _lane_gather(v, jnp.full((16,), hd, dtype=jnp.int32))


def _pre1_body(x_ref, wt_ref, ams_ref, amd_ref, h_ref, as_ref, ad_ref):
  h = jnp.dot(x_ref[...], wt_ref[...], preferred_element_type=jnp.float32)
  h_ref[...] = h
  as_ref[...] = jnp.dot(h, ams_ref[...], preferred_element_type=jnp.float32)
  ad_ref[...] = jnp.dot(h, amd_ref[...], preferred_element_type=jnp.float32)


def _mid_body(acc_ref, den_ref, b1_ref, wt_ref, ams_ref, amd_ref,
              h2_ref, as_ref, ad_ref):
  acc = acc_ref[0] + acc_ref[1]
  den = den_ref[0] + den_ref[1]
  h = acc / (den + 1e-16) + b1_ref[...]
  h = jnp.where(h > 0, h, 0.2 * (jnp.exp(h) - 1.0))
  h2 = jnp.dot(h, wt_ref[...], preferred_element_type=jnp.float32)
  h2_ref[:, :C2] = h2
  h2_ref[:, C2:] = jnp.zeros_like(h2)
  as_ref[...] = jnp.dot(h2, ams_ref[...], preferred_element_type=jnp.float32)
  ad_ref[...] = jnp.dot(h2, amd_ref[...], preferred_element_type=jnp.float32)


def _fin_body(acc_ref, den_ref, b2_ref, out_ref):
  acc = acc_ref[0] + acc_ref[1]
  den = den_ref[0] + den_ref[1]
  out_ref[...] = acc[:, :C2] / (den[:, :C2] + 1e-16) + b2_ref[...]


def _make_edge_pass(nheads):
  head_of = [min(j, nheads - 1) for j in range(8)]
  mesh = plsc.VectorSubcoreMesh(core_axis_name="c", subcore_axis_name="s")
  cp = pltpu.CompilerParams()
  if "needs_layout_passes" in pltpu.CompilerParams.__dataclass_fields__:
    cp = dataclasses.replace(cp, needs_layout_passes=False)

  @functools.partial(
      pl.kernel,
      out_type=(jax.ShapeDtypeStruct((2, NP, 128), jnp.float32),
                jax.ShapeDtypeStruct((2, NP, 128), jnp.float32)),
      mesh=mesh,
      compiler_params=cp,
      scratch_types=(
          [pltpu.VMEM((K,), jnp.int32)] * 6 +
          [pltpu.VMEM((2 * K,), jnp.int32)] * 2 +
          [pltpu.VMEM((2 * K, 128), jnp.float32)] * 2 +
          [pltpu.VMEM((K, 128), jnp.float32)] * 4 +
          [pltpu.VMEM_SHARED((NP_ACC, 128), jnp.float32)] +
          [pltpu.SemaphoreType.DMA] * 5
      ),
  )
  def edge_pass(h_hbm, as_hbm, ad_hbm, src_hbm, dst_hbm, acc_hbm, den_hbm,
                si0, si1, si2, di0, di1, di2, ci0, ci1,
                cb0, cb1, as0, as1, ad0, ad1,
                acc_sh, is0, is1, is2, gs0, gs1):
    SIDX, DIDX, CIDX = [si0, si1, si2], [di0, di1, di2], [ci0, ci1]
    AS, AD, CB = [as0, as1], [ad0, ad1], [cb0, cb1]
    ISEM, GSEM = [is0, is1, is2], [gs0, gs1]

    cid = lax.axis_index("c")
    sid = lax.axis_index("s")
    wid = cid * 16 + sid
    lane = lax.iota(jnp.int32, 16)
    zero16 = jnp.zeros((16,), jnp.float32)

    def issue_idx(cidx, ib):
      base = (wid * CHUNKS + cidx) * K
      pltpu.async_copy(src_hbm.at[pl.ds(base, K)], SIDX[ib], ISEM[ib])
      pltpu.async_copy(dst_hbm.at[pl.ds(base, K)], DIDX[ib], ISEM[ib])

    def wait_idx(ib):
      pltpu.make_async_copy(src_hbm.at[pl.ds(0, K)], SIDX[ib], ISEM[ib]).wait()
      pltpu.make_async_copy(dst_hbm.at[pl.ds(0, K)], DIDX[ib], ISEM[ib]).wait()

    def issue_gathers(db, ib):
      pltpu.async_copy(as_hbm.at[SIDX[ib]], AS[db], GSEM[db])
      pltpu.async_copy(ad_hbm.at[DIDX[ib]], AD[db], GSEM[db])
      pltpu.async_copy(h_hbm.at[SIDX[ib]], CB[db].at[pl.ds(0, K)], GSEM[db])

    def wait_gathers(db, ib):
      pltpu.make_async_copy(as_hbm.at[SIDX[ib]], AS[db], GSEM[db]).wait()
      pltpu.make_async_copy(ad_hbm.at[DIDX[ib]], AD[db], GSEM[db]).wait()
      pltpu.make_async_copy(h_hbm.at[SIDX[ib]], CB[db].at[pl.ds(0, K)],
                            GSEM[db]).wait()

    def sync_scatters(db, ib):
      for i in _W16:
        d16 = DIDX[ib][pl.ds(i, 16)]
        CIDX[db][pl.ds(i, 16)] = d16
        CIDX[db][pl.ds(K + i, 16)] = (
            lax.shift_right_logical(d16, 4) + NP)
      pltpu.sync_copy(CB[db], acc_sh.at[CIDX[db]], add=True)

    def compute(db, ib):
      as_b, ad_b, c_b, didx = AS[db], AD[db], CB[db], DIDX[ib]

      @pl.loop(0, K)
      def _(e):
        al = as_b[e, pl.ds(0, 16)] + ad_b[e, pl.ds(0, 16)]
        al = jnp.where(al > 0, al, al * 0.2)
        w = jnp.exp(al)
        dv = plsc.load_gather(didx, [jnp.full((16,), e, jnp.int32)])
        deq = lane == (dv & 15)
        for j in range(8):
          sl = pl.ds(j * 16, 16)
          wb = _lane_bcast(w, head_of[j])
          c_b[e, sl] = c_b[e, sl] * wb
          if head_of[j] == j:
            c_b[K + e, sl] = jnp.where(deq, wb, 0.0)

    @pl.loop(0, 2 * K)
    def _(i):
      @pl.loop(0, 128, step=16)
      def _(j):
        cb0[i, pl.ds(j, 16)] = zero16
        cb1[i, pl.ds(j, 16)] = zero16

    row0 = sid * ROWS_PER_SUB
    drow0 = sid * DROWS_PER_SUB

    @pl.loop(0, ROWS_PER_SUB, step=2 * K)
    def _(r):
      pltpu.sync_copy(cb0, acc_sh.at[pl.ds(row0 + r, 2 * K)])

    pltpu.sync_copy(cb0.at[pl.ds(0, DROWS_PER_SUB)],
                    acc_sh.at[pl.ds(NP + drow0, DROWS_PER_SUB)])

    plsc.subcore_barrier()

    issue_idx(0, 0)
    issue_idx(1, 1)
    wait_idx(0)
    issue_gathers(0, 0)

    wait_gathers(0, 0)
    issue_idx(2, 2)
    wait_idx(1)
    issue_gathers(1, 1)
    compute(0, 0)
    sync_scatters(0, 0)

    @pl.loop(0, (CHUNKS - 1) // 6)
    def _(t):
      g0 = 1 + t * 6
      for u in range(6):
        g = g0 + u
        db, dp = (1 + u) & 1, u & 1
        ib_g, ib_n1, ib_n2 = (1 + u) % 3, (2 + u) % 3, u % 3
        wait_gathers(db, ib_g)
        issue_idx(g + 2, ib_n2)
        wait_idx(ib_n1)
        issue_gathers(dp, ib_n1)
        compute(db, ib_g)
        sync_scatters(db, ib_g)

    wait_gathers(1, 1)
    wait_idx(2)
    plsc.subcore_barrier()

    pltpu.sync_copy(acc_sh.at[pl.ds(row0, ROWS_PER_SUB)],
                    acc_hbm.at[cid].at[pl.ds(row0, ROWS_PER_SUB)])

    pltpu.sync_copy(acc_sh.at[pl.ds(NP + drow0, DROWS_PER_SUB)],
                    ad0.at[pl.ds(0, DROWS_PER_SUB)])

    @pl.loop(0, ROWS_PER_SUB, step=2 * K)
    def _(r):
      @pl.loop(0, 2 * K)
      def _(u):
        nl = r + u
        rr = lax.shift_right_logical(nl, 4)
        m = jnp.full((16,), nl & 15, jnp.int32)
        for j in range(8):
          q = ad0[rr, pl.ds(head_of[j] * 16, 16)]
          cb0[u, pl.ds(j * 16, 16)] = _lane_gather(q, m)

      pltpu.sync_copy(cb0, den_hbm.at[cid].at[pl.ds(row0 + r, 2 * K)])

  return edge_pass


_edge_pass1 = _make_edge_pass(H1)
_edge_pass2 = _make_edge_pass(H2)


def _att_mat(att, D):
  nheads = att.shape[1]
  cdim = D // nheads
  cols = jnp.repeat(jnp.arange(nheads, dtype=jnp.int32), cdim)
  m = jnp.zeros((D, 128), jnp.float32)
  return m.at[jnp.arange(D), cols].set(att.reshape(D))


def kernel(x, edge_index, W1, att_src1, att_dst1, b1, W2, att_src2, att_dst2,
           b2):
  loop = jnp.arange(N, dtype=edge_index.dtype)
  src = jnp.concatenate([edge_index[0], loop]).astype(jnp.int32)
  dst = jnp.concatenate([edge_index[1], loop]).astype(jnp.int32)
  pad = jnp.full((EP_ALLOC - E_LOOP,), N, jnp.int32)
  src = jnp.concatenate([src, pad])
  dst = jnp.concatenate([dst, pad])

  x_p = jnp.pad(x, ((0, NP - N), (0, 0)))
  ams1 = _att_mat(att_src1, D1)
  amd1 = _att_mat(att_dst1, D1)
  ams2 = _att_mat(att_src2, C2)
  amd2 = _att_mat(att_dst2, C2)

  grid = NP // BLK
  row_spec = lambda w: pl.BlockSpec((BLK, w), lambda i: (i, 0))
  full_spec = lambda a, b: pl.BlockSpec((a, b), lambda i: (0, 0))
  pair_spec = pl.BlockSpec((2, BLK, 128), lambda i: (0, i, 0))

  h1, as1, ad1 = pl.pallas_call(
      _pre1_body,
      grid=(grid,),
      in_specs=[row_spec(F_IN), full_spec(F_IN, D1), full_spec(D1, 128),
                full_spec(D1, 128)],
      out_specs=[row_spec(D1), row_spec(128), row_spec(128)],
      out_shape=[jax.ShapeDtypeStruct((NP, D1), jnp.float32),
                 jax.ShapeDtypeStruct((NP, 128), jnp.float32),
                 jax.ShapeDtypeStruct((NP, 128), jnp.float32)],
  )(x_p, W1.T, ams1, amd1)

  acc1, den1 = _edge_pass1(h1, as1, ad1, src, dst)

  h2, as2, ad2 = pl.pallas_call(
      _mid_body,
      grid=(grid,),
      in_specs=[pair_spec, pair_spec, full_spec(1, D1), full_spec(D1, C2),
                full_spec(C2, 128), full_spec(C2, 128)],
      out_specs=[row_spec(128), row_spec(128), row_spec(128)],
      out_shape=[jax.ShapeDtypeStruct((NP, 128), jnp.float32),
                 jax.ShapeDtypeStruct((NP, 128), jnp.float32),
                 jax.ShapeDtypeStruct((NP, 128), jnp.float32)],
  )(acc1, den1, b1.reshape(1, D1), W2.T, ams2, amd2)

  acc2, den2 = _edge_pass2(h2, as2, ad2, src, dst)

  out = pl.pallas_call(
      _fin_body,
      grid=(grid,),
      in_specs=[pair_spec, pair_spec, full_spec(1, C2)],
      out_specs=pl.BlockSpec((BLK, C2), lambda i: (i, 0)),
      out_shape=jax.ShapeDtypeStruct((NP, C2), jnp.float32),
  )(acc2, den2, b2.reshape(1, C2))

  return out[:N]


# speedup vs baseline: 3.6845x; 1.0056x over previous
import dataclasses
import functools

import jax
import jax.numpy as jnp
from jax import lax
from jax.experimental import pallas as pl
from jax.experimental.pallas import tpu as pltpu
from jax.experimental.pallas import tpu_sc as plsc

N = 10000
NP = 10240
F_IN = 128
H1, C1 = 8, 16
D1 = H1 * C1
H2, C2 = 1, 64
E_RAW = 320000
E_LOOP = E_RAW + N
K = 40
NWORK = 32
CHUNKS = 259
EP = CHUNKS * K * NWORK
EP_ALLOC = EP + 2 * K
ROWS_PER_SUB = NP // 16
DROWS_PER_SUB = ROWS_PER_SUB // 16
NP_ACC = NP + NP // 16
BLK = 1024
_W16 = (0, 16, 24)

_GD = lax.GatherDimensionNumbers(
    offset_dims=(), collapsed_slice_dims=(0,), start_index_map=(0,))


def _lane_gather(v, idx):
  return lax.gather(v, idx.reshape(16, 1), _GD, (1,),
                    mode=lax.GatherScatterMode.PROMISE_IN_BOUNDS)


def _lane_bcast(v, hd):
  return _lane_gather(v, jnp.full((16,), hd, dtype=jnp.int32))


def _pre1_body(x_ref, wt_ref, ams_ref, amd_ref, h_ref, as_ref, ad_ref):
  h = jnp.dot(x_ref[...], wt_ref[...], preferred_element_type=jnp.float32)
  h_ref[...] = h
  as_ref[...] = jnp.dot(h, ams_ref[...], preferred_element_type=jnp.float32)
  ad_ref[...] = jnp.dot(h, amd_ref[...], preferred_element_type=jnp.float32)


def _mid_body(acc_ref, den_ref, b1_ref, wt_ref, ams_ref, amd_ref,
              h2_ref, as_ref, ad_ref):
  acc = acc_ref[0] + acc_ref[1]
  den = den_ref[0] + den_ref[1]
  h = acc / (den + 1e-16) + b1_ref[...]
  h = jnp.where(h > 0, h, 0.2 * (jnp.exp(h) - 1.0))
  h2 = jnp.dot(h, wt_ref[...], preferred_element_type=jnp.float32)
  h2_ref[:, :C2] = h2
  h2_ref[:, C2:] = jnp.zeros_like(h2)
  as_ref[...] = jnp.dot(h2, ams_ref[...], preferred_element_type=jnp.float32)
  ad_ref[...] = jnp.dot(h2, amd_ref[...], preferred_element_type=jnp.float32)


def _fin_body(acc_ref, den_ref, b2_ref, out_ref):
  acc = acc_ref[0] + acc_ref[1]
  den = den_ref[0] + den_ref[1]
  out_ref[...] = acc[:, :C2] / (den[:, :C2] + 1e-16) + b2_ref[...]


def _make_edge_pass(nheads):
  head_of = [min(j, nheads - 1) for j in range(8)]
  mesh = plsc.VectorSubcoreMesh(core_axis_name="c", subcore_axis_name="s")
  cp = pltpu.CompilerParams()
  if "needs_layout_passes" in pltpu.CompilerParams.__dataclass_fields__:
    cp = dataclasses.replace(cp, needs_layout_passes=False)

  @functools.partial(
      pl.kernel,
      out_type=(jax.ShapeDtypeStruct((2, NP, 128), jnp.float32),
                jax.ShapeDtypeStruct((2, NP, 128), jnp.float32)),
      mesh=mesh,
      compiler_params=cp,
      scratch_types=(
          [pltpu.VMEM((K,), jnp.int32)] * 6 +
          [pltpu.VMEM((2 * K,), jnp.int32)] * 2 +
          [pltpu.VMEM((2 * K, 128), jnp.float32)] * 2 +
          [pltpu.VMEM((K, 128), jnp.float32)] * 4 +
          [pltpu.VMEM_SHARED((NP_ACC, 128), jnp.float32)] +
          [pltpu.SemaphoreType.DMA] * 5
      ),
  )
  def edge_pass(h_hbm, as_hbm, ad_hbm, src_hbm, dst_hbm, acc_hbm, den_hbm,
                si0, si1, si2, di0, di1, di2, ci0, ci1,
                cb0, cb1, as0, as1, ad0, ad1,
                acc_sh, is0, is1, is2, gs0, gs1):
    SIDX, DIDX, CIDX = [si0, si1, si2], [di0, di1, di2], [ci0, ci1]
    AS, AD, CB = [as0, as1], [ad0, ad1], [cb0, cb1]
    ISEM, GSEM = [is0, is1, is2], [gs0, gs1]

    cid = lax.axis_index("c")
    sid = lax.axis_index("s")
    wid = cid * 16 + sid
    lane = lax.iota(jnp.int32, 16)
    zero16 = jnp.zeros((16,), jnp.float32)

    def issue_idx(cidx, ib):
      base = (wid * CHUNKS + cidx) * K
      pltpu.async_copy(src_hbm.at[pl.ds(base, K)], SIDX[ib], ISEM[ib])
      pltpu.async_copy(dst_hbm.at[pl.ds(base, K)], DIDX[ib], ISEM[ib])

    def wait_idx(ib):
      pltpu.make_async_copy(src_hbm.at[pl.ds(0, K)], SIDX[ib], ISEM[ib]).wait()
      pltpu.make_async_copy(dst_hbm.at[pl.ds(0, K)], DIDX[ib], ISEM[ib]).wait()

    def issue_gathers(db, ib):
      pltpu.async_copy(as_hbm.at[SIDX[ib]], AS[db], GSEM[db])
      pltpu.async_copy(ad_hbm.at[DIDX[ib]], AD[db], GSEM[db])
      pltpu.async_copy(h_hbm.at[SIDX[ib]], CB[db].at[pl.ds(0, K)], GSEM[db])

    def wait_gathers(db, ib):
      pltpu.make_async_copy(as_hbm.at[SIDX[ib]], AS[db], GSEM[db]).wait()
      pltpu.make_async_copy(ad_hbm.at[DIDX[ib]], AD[db], GSEM[db]).wait()
      pltpu.make_async_copy(h_hbm.at[SIDX[ib]], CB[db].at[pl.ds(0, K)],
                            GSEM[db]).wait()

    def sync_scatters(db, ib):
      for i in _W16:
        d16 = DIDX[ib][pl.ds(i, 16)]
        CIDX[db][pl.ds(i, 16)] = d16
        CIDX[db][pl.ds(K + i, 16)] = (
            lax.shift_right_logical(d16, 4) + NP)
      pltpu.sync_copy(CB[db], acc_sh.at[CIDX[db]], add=True)

    def compute(db, ib):
      as_b, ad_b, c_b, didx = AS[db], AD[db], CB[db], DIDX[ib]

      @pl.loop(0, K, step=2)
      def _(e0):
        for e in (e0, e0 + 1):
          al = as_b[e, pl.ds(0, 16)] + ad_b[e, pl.ds(0, 16)]
          al = jnp.where(al > 0, al, al * 0.2)
          w = jnp.exp(al)
          dv = plsc.load_gather(didx, [jnp.full((16,), e, jnp.int32)])
          deq = lane == (dv & 15)
          for j in range(8):
            sl = pl.ds(j * 16, 16)
            wb = _lane_bcast(w, head_of[j])
            c_b[e, sl] = c_b[e, sl] * wb
            if head_of[j] == j:
              c_b[K + e, sl] = jnp.where(deq, wb, 0.0)

    @pl.loop(0, 2 * K)
    def _(i):
      @pl.loop(0, 128, step=16)
      def _(j):
        cb0[i, pl.ds(j, 16)] = zero16
        cb1[i, pl.ds(j, 16)] = zero16

    row0 = sid * ROWS_PER_SUB
    drow0 = sid * DROWS_PER_SUB

    @pl.loop(0, ROWS_PER_SUB, step=2 * K)
    def _(r):
      pltpu.sync_copy(cb0, acc_sh.at[pl.ds(row0 + r, 2 * K)])

    pltpu.sync_copy(cb0.at[pl.ds(0, DROWS_PER_SUB)],
                    acc_sh.at[pl.ds(NP + drow0, DROWS_PER_SUB)])

    plsc.subcore_barrier()

    issue_idx(0, 0)
    issue_idx(1, 1)
    wait_idx(0)
    issue_gathers(0, 0)

    wait_gathers(0, 0)
    issue_idx(2, 2)
    wait_idx(1)
    issue_gathers(1, 1)
    compute(0, 0)
    sync_scatters(0, 0)

    @pl.loop(0, (CHUNKS - 1) // 6)
    def _(t):
      g0 = 1 + t * 6
      for u in range(6):
        g = g0 + u
        db, dp = (1 + u) & 1, u & 1
        ib_g, ib_n1, ib_n2 = (1 + u) % 3, (2 + u) % 3, u % 3
        wait_gathers(db, ib_g)
        issue_idx(g + 2, ib_n2)
        wait_idx(ib_n1)
        issue_gathers(dp, ib_n1)
        compute(db, ib_g)
        sync_scatters(db, ib_g)

    wait_gathers(1, 1)
    wait_idx(2)
    plsc.subcore_barrier()

    pltpu.sync_copy(acc_sh.at[pl.ds(row0, ROWS_PER_SUB)],
                    acc_hbm.at[cid].at[pl.ds(row0, ROWS_PER_SUB)])

    pltpu.sync_copy(acc_sh.at[pl.ds(NP + drow0, DROWS_PER_SUB)],
                    ad0.at[pl.ds(0, DROWS_PER_SUB)])

    @pl.loop(0, ROWS_PER_SUB, step=2 * K)
    def _(r):
      @pl.loop(0, 2 * K)
      def _(u):
        nl = r + u
        rr = lax.shift_right_logical(nl, 4)
        m = jnp.full((16,), nl & 15, jnp.int32)
        for j in range(8):
          q = ad0[rr, pl.ds(head_of[j] * 16, 16)]
          cb0[u, pl.ds(j * 16, 16)] = _lane_gather(q, m)

      pltpu.sync_copy(cb0, den_hbm.at[cid].at[pl.ds(row0 + r, 2 * K)])

  return edge_pass


_edge_pass1 = _make_edge_pass(H1)
_edge_pass2 = _make_edge_pass(H2)


def _att_mat(att, D):
  nheads = att.shape[1]
  cdim = D // nheads
  cols = jnp.repeat(jnp.arange(nheads, dtype=jnp.int32), cdim)
  m = jnp.zeros((D, 128), jnp.float32)
  return m.at[jnp.arange(D), cols].set(att.reshape(D))


def kernel(x, edge_index, W1, att_src1, att_dst1, b1, W2, att_src2, att_dst2,
           b2):
  loop = jnp.arange(N, dtype=edge_index.dtype)
  src = jnp.concatenate([edge_index[0], loop]).astype(jnp.int32)
  dst = jnp.concatenate([edge_index[1], loop]).astype(jnp.int32)
  pad = jnp.full((EP_ALLOC - E_LOOP,), N, jnp.int32)
  src = jnp.concatenate([src, pad])
  dst = jnp.concatenate([dst, pad])

  x_p = jnp.pad(x, ((0, NP - N), (0, 0)))
  ams1 = _att_mat(att_src1, D1)
  amd1 = _att_mat(att_dst1, D1)
  ams2 = _att_mat(att_src2, C2)
  amd2 = _att_mat(att_dst2, C2)

  grid = NP // BLK
  row_spec = lambda w: pl.BlockSpec((BLK, w), lambda i: (i, 0))
  full_spec = lambda a, b: pl.BlockSpec((a, b), lambda i: (0, 0))
  pair_spec = pl.BlockSpec((2, BLK, 128), lambda i: (0, i, 0))

  h1, as1, ad1 = pl.pallas_call(
      _pre1_body,
      grid=(grid,),
      in_specs=[row_spec(F_IN), full_spec(F_IN, D1), full_spec(D1, 128),
                full_spec(D1, 128)],
      out_specs=[row_spec(D1), row_spec(128), row_spec(128)],
      out_shape=[jax.ShapeDtypeStruct((NP, D1), jnp.float32),
                 jax.ShapeDtypeStruct((NP, 128), jnp.float32),
                 jax.ShapeDtypeStruct((NP, 128), jnp.float32)],
  )(x_p, W1.T, ams1, amd1)

  acc1, den1 = _edge_pass1(h1, as1, ad1, src, dst)

  h2, as2, ad2 = pl.pallas_call(
      _mid_body,
      grid=(grid,),
      in_specs=[pair_spec, pair_spec, full_spec(1, D1), full_spec(D1, C2),
                full_spec(C2, 128), full_spec(C2, 128)],
      out_specs=[row_spec(128), row_spec(128), row_spec(128)],
      out_shape=[jax.ShapeDtypeStruct((NP, 128), jnp.float32),
                 jax.ShapeDtypeStruct((NP, 128), jnp.float32),
                 jax.ShapeDtypeStruct((NP, 128), jnp.float32)],
  )(acc1, den1, b1.reshape(1, D1), W2.T, ams2, amd2)

  acc2, den2 = _edge_pass2(h2, as2, ad2, src, dst)

  out = pl.pallas_call(
      _fin_body,
      grid=(grid,),
      in_specs=[pair_spec, pair_spec, full_spec(1, C2)],
      out_specs=pl.BlockSpec((BLK, C2), lambda i: (i, 0)),
      out_shape=jax.ShapeDtypeStruct((NP, C2), jnp.float32),
  )(acc2, den2, b2.reshape(1, C2))

  return out[:N]
